# baseline (speedup 1.0000x reference)
import functools

import jax
import jax.numpy as jnp
from jax import lax
from jax.experimental import pallas as pl
from jax.experimental.pallas import tpu as pltpu
from jax.experimental.pallas import tpu_sc as plsc

NC = 2
NS = 16
CHB = 48
IB = 6


def _hext_body(c, cw, logits_ref, h_ref, out_ref):
    lg = logits_ref[...]
    h = h_ref[...]
    m = jnp.max(lg, axis=1, keepdims=True)
    iota_c = lax.broadcasted_iota(jnp.int32, lg.shape, 1)
    cls = jnp.min(jnp.where(lg == m, iota_c, c), axis=1, keepdims=True)
    iota_w = lax.broadcasted_iota(jnp.int32, (lg.shape[0], cw), 1)
    oh = (iota_w == cls).astype(jnp.float32)
    out_ref[...] = jnp.concatenate([h, oh], axis=1)


def _build_hext(h, logits, cw, interpret=False):
    n, d = h.shape
    c = logits.shape[1]
    br = 1000 if n % 1000 == 0 else n
    grid = (n // br,)
    return pl.pallas_call(
        functools.partial(_hext_body, c, cw),
        grid=grid,
        in_specs=[
            pl.BlockSpec((br, c), lambda i: (i, 0)),
            pl.BlockSpec((br, d), lambda i: (i, 0)),
        ],
        out_specs=pl.BlockSpec((br, d + cw), lambda i: (i, 0)),
        out_shape=jax.ShapeDtypeStruct((n, d + cw), jnp.float32),
        interpret=interpret,
    )(logits, h)


def _edge_sc(n, d, cw, nch, rt, hext, srcp, dstp, zrows):
    w = d + cw
    rows_per_tile = rt // NS
    out_rows = (n // (NS * 8)) * 8
    tail = n - NS * out_rows

    mesh = plsc.VectorSubcoreMesh(core_axis_name="c", subcore_axis_name="s")

    @functools.partial(
        pl.kernel,
        out_type=(
            jax.ShapeDtypeStruct((NC, n, d), jnp.float32),
            jax.ShapeDtypeStruct((NC, n, cw), jnp.float32),
        ),
        mesh=mesh,
        scratch_types=[
            pltpu.VMEM_SHARED((rt, w), jnp.float32),
            [pltpu.VMEM((CHB, w), jnp.float32) for _ in range(2)],
            [pltpu.VMEM((IB, CHB), jnp.int32) for _ in range(2)],
            [pltpu.VMEM((IB, CHB), jnp.int32) for _ in range(2)],
            [pltpu.SemaphoreType.DMA for _ in range(2)],
            [pltpu.SemaphoreType.DMA for _ in range(2)],
        ],
        compiler_params=pltpu.CompilerParams(use_tc_tiling_on_sc=False),
    )
    def k(hext_hbm, src2_hbm, dst2_hbm, zrows_hbm,
          agg_out, cnt_out, acc_sh, rows, srcb, dstb, isem, gsem):
        c = lax.axis_index("c")
        s = lax.axis_index("s")
        wid = c * NS + s

        pltpu.sync_copy(zrows_hbm, acc_sh.at[pl.ds(s * rows_per_tile,
                                                   rows_per_tile)])
        plsc.subcore_barrier()

        nsc = nch // IB
        rbase = wid * nch

        def idx_start(g, r):
            po = rbase + g * IB
            pltpu.async_copy(src2_hbm.at[pl.ds(po, IB)], srcb[r], isem[r])
            pltpu.async_copy(dst2_hbm.at[pl.ds(po, IB)], dstb[r], isem[r])

        def idx_wait(r):
            pltpu.make_async_copy(src2_hbm.at[pl.ds(0, IB)],
                                  srcb[r], isem[r]).wait()
            pltpu.make_async_copy(dst2_hbm.at[pl.ds(0, IB)],
                                  dstb[r], isem[r]).wait()

        def gather_start(r, j, rb):
            pltpu.async_copy(hext_hbm.at[srcb[r].at[j]], rows[rb], gsem[rb])

        def gather_wait(r, j, rb):
            pltpu.make_async_copy(hext_hbm.at[srcb[r].at[j]],
                                  rows[rb], gsem[rb]).wait()

        idx_start(0, 0)
        idx_wait(0)
        gather_start(0, 0, 0)

        def pair_body(gp, carry):
            for gg in range(2):
                g = gp * 2 + gg
                r, o = gg, 1 - gg
                for j in range(IB):
                    rb = j % 2

                    if j == 0:
                        @pl.when(g + 1 < nsc)
                        def _():
                            idx_start(g + 1, o)

                    if j < IB - 1:
                        gather_start(r, j + 1, 1 - rb)
                    else:
                        @pl.when(g + 1 < nsc)
                        def _():
                            idx_wait(o)
                            gather_start(o, 0, 1 - rb)

                    gather_wait(r, j, rb)
                    pltpu.sync_copy(rows[rb], acc_sh.at[dstb[r].at[j]],
                                    add=True)
            return carry

        lax.fori_loop(0, nsc // 2, pair_body, 0, unroll=False)
        plsc.subcore_barrier()

        rb = s * out_rows
        pltpu.sync_copy(acc_sh.at[pl.ds(rb, out_rows), pl.ds(0, d)],
                        agg_out.at[c, pl.ds(rb, out_rows)])
        pltpu.sync_copy(acc_sh.at[pl.ds(rb, out_rows), pl.ds(d, cw)],
                        cnt_out.at[c, pl.ds(rb, out_rows)])
        if tail:
            tb = NS * out_rows

            @pl.when(s == 0)
            def _():
                pltpu.sync_copy(acc_sh.at[pl.ds(tb, tail), pl.ds(0, d)],
                                agg_out.at[c, pl.ds(tb, tail)])
                pltpu.sync_copy(acc_sh.at[pl.ds(tb, tail), pl.ds(d, cw)],
                                cnt_out.at[c, pl.ds(tb, tail)])

    return k(hext, srcp, dstp, zrows)


def _gate_body(cnt_ref, oh_ref, oldz_ref, t1_ref, t2_ref, z_ref, gate_ref):
    counts = cnt_ref[0] + cnt_ref[1]
    oh = oh_ref[...]
    deg = jnp.sum(counts, axis=1, keepdims=True)
    match = jnp.sum(counts * oh, axis=1, keepdims=True)
    f1 = match / deg

    gc = jnp.sum(counts, axis=0, keepdims=True)
    present = gc > 0.0
    cnts_p = jnp.clip(counts / deg, 1e-5, None)
    ent = cnts_p * jnp.log(cnts_p)
    f2 = -jnp.sum(jnp.where(present, ent, 0.0), axis=1, keepdims=True)

    def _ln(x):
        m = jnp.mean(x)
        v = jnp.mean((x - m) ** 2)
        return (x - m) / jnp.sqrt(v + 1e-5)

    nf1 = _ln(f1)
    nf2 = _ln(f2)
    t1 = t1_ref[0, 0]
    t2 = t2_ref[0, 0]
    z = jax.nn.sigmoid(-(nf1 - t1)) * jax.nn.sigmoid(-(nf2 - t2))
    z_ref[...] = z
    gate_ref[...] = jnp.minimum(oldz_ref[...], z)


def _gates(cnt_part, oh, old_z, tau_1, tau_2, interpret=False):
    n = oh.shape[0]
    return pl.pallas_call(
        _gate_body,
        out_shape=(
            jax.ShapeDtypeStruct((n, 1), jnp.float32),
            jax.ShapeDtypeStruct((n, 1), jnp.float32),
        ),
        interpret=interpret,
    )(cnt_part, oh, old_z.reshape(n, 1), tau_1.reshape(1, 1),
      tau_2.reshape(1, 1))


def _newh_body(h_ref, agg_ref, gate_ref, norm_ref, out_ref):
    agg = agg_ref[0] + agg_ref[1]
    normagg = jax.nn.relu(agg * norm_ref[...])
    out_ref[...] = h_ref[...] + gate_ref[...] * normagg


def _new_h(h, agg_part, gate, norm, interpret=False):
    n, d = h.shape
    br = 1000 if n % 1000 == 0 else n
    grid = (n // br,)
    return pl.pallas_call(
        _newh_body,
        grid=grid,
        in_specs=[
            pl.BlockSpec((br, d), lambda i: (i, 0)),
            pl.BlockSpec((NC, br, d), lambda i: (0, i, 0)),
            pl.BlockSpec((br, 1), lambda i: (i, 0)),
            pl.BlockSpec((br, 1), lambda i: (i, 0)),
        ],
        out_specs=pl.BlockSpec((br, d), lambda i: (i, 0)),
        out_shape=jax.ShapeDtypeStruct((n, d), jnp.float32),
        interpret=interpret,
    )(h, agg_part, gate, norm.reshape(n, 1))


@jax.jit
def kernel(h, logits, old_z, norm, tau_1, tau_2, edge_index):
    n, d = h.shape
    c = logits.shape[1]
    cw = ((c + 15) // 16) * 16
    e = edge_index.shape[1]

    nt = NC * NS
    nch = -(-e // (nt * CHB))
    nch = -(-nch // (2 * IB)) * 2 * IB
    ep = nt * CHB * nch
    pad = ep - e

    src = edge_index[0]
    dst = edge_index[1]
    if pad:
        pad_dst = n + (jnp.arange(pad, dtype=jnp.int32) % 8)
        src = jnp.concatenate([src, jnp.zeros((pad,), jnp.int32)])
        dst = jnp.concatenate([dst, pad_dst])

    rt = -(-(n + 8) // (NS * 8)) * NS * 8
    zrows = jnp.zeros((rt // NS, d + cw), jnp.float32)

    hext = _build_hext(h, logits, cw)
    agg_part, cnt_part = _edge_sc(n, d, cw, nch, rt, hext,
                                  src.reshape(-1, CHB), dst.reshape(-1, CHB),
                                  zrows)

    oh = lax.slice(hext, (0, d), (n, d + cw))
    z, gate = _gates(cnt_part, oh, old_z, tau_1, tau_2)
    new_h = _new_h(h, agg_part, gate, norm)
    return new_h, z.reshape(n)


# speedup vs baseline: 2.0504x; 2.0504x over previous
import functools

import jax
import jax.numpy as jnp
from jax import lax
from jax.experimental import pallas as pl
from jax.experimental.pallas import tpu as pltpu
from jax.experimental.pallas import tpu_sc as plsc

NC = 2
NS = 16
CHB = 48


def _hext_body(c, cw, logits_ref, h_ref, out_ref):
    lg = logits_ref[...]
    h = h_ref[...]
    m = jnp.max(lg, axis=1, keepdims=True)
    iota_c = lax.broadcasted_iota(jnp.int32, lg.shape, 1)
    cls = jnp.min(jnp.where(lg == m, iota_c, c), axis=1, keepdims=True)
    iota_w = lax.broadcasted_iota(jnp.int32, (lg.shape[0], cw), 1)
    oh = (iota_w == cls).astype(jnp.float32)
    out_ref[...] = jnp.concatenate([h, oh], axis=1)


def _build_hext(h, logits, cw, interpret=False):
    n, d = h.shape
    c = logits.shape[1]
    br = 1000 if n % 1000 == 0 else n
    grid = (n // br,)
    return pl.pallas_call(
        functools.partial(_hext_body, c, cw),
        grid=grid,
        in_specs=[
            pl.BlockSpec((br, c), lambda i: (i, 0)),
            pl.BlockSpec((br, d), lambda i: (i, 0)),
        ],
        out_specs=pl.BlockSpec((br, d + cw), lambda i: (i, 0)),
        out_shape=jax.ShapeDtypeStruct((n, d + cw), jnp.float32),
        interpret=interpret,
    )(logits, h)


def _edge_sc(n, d, cw, nch, rt, hext, srcp, dstp, zrows):
    w = d + cw
    rows_per_tile = rt // NS
    out_rows = (n // (NS * 8)) * 8
    tail = n - NS * out_rows

    mesh = plsc.VectorSubcoreMesh(core_axis_name="c", subcore_axis_name="s")

    @functools.partial(
        pl.kernel,
        out_type=(
            jax.ShapeDtypeStruct((NC, n, d), jnp.float32),
            jax.ShapeDtypeStruct((NC, n, cw), jnp.float32),
        ),
        mesh=mesh,
        scratch_types=[
            pltpu.VMEM_SHARED((rt, w), jnp.float32),
            [pltpu.VMEM((CHB, w), jnp.float32) for _ in range(2)],
            [pltpu.VMEM((CHB,), jnp.int32) for _ in range(3)],
            [pltpu.VMEM((CHB,), jnp.int32) for _ in range(3)],
            [pltpu.SemaphoreType.DMA for _ in range(3)],
            [pltpu.SemaphoreType.DMA for _ in range(2)],
        ],
        compiler_params=pltpu.CompilerParams(use_tc_tiling_on_sc=False),
    )
    def k(hext_hbm, src_hbm, dst_hbm, zrows_hbm,
          agg_out, cnt_out, acc_sh, rows, srcs, dsts, isem, gsem):
        c = lax.axis_index("c")
        s = lax.axis_index("s")
        wid = c * NS + s

        pltpu.sync_copy(zrows_hbm, acc_sh.at[pl.ds(s * rows_per_tile,
                                                   rows_per_tile)])
        plsc.subcore_barrier()

        ebase = wid * (nch * CHB)

        def idx_start(kk, r):
            off = pl.multiple_of(ebase + kk * CHB, CHB)
            pltpu.async_copy(src_hbm.at[pl.ds(off, CHB)], srcs[r], isem[r])
            pltpu.async_copy(dst_hbm.at[pl.ds(off, CHB)], dsts[r], isem[r])

        def idx_wait(r):
            pltpu.make_async_copy(src_hbm.at[pl.ds(0, CHB)],
                                  srcs[r], isem[r]).wait()
            pltpu.make_async_copy(dst_hbm.at[pl.ds(0, CHB)],
                                  dsts[r], isem[r]).wait()

        def gather_start(r, rb):
            pltpu.async_copy(hext_hbm.at[srcs[r]], rows[rb], gsem[rb])

        def gather_wait(r, rb):
            pltpu.make_async_copy(hext_hbm.at[srcs[r]],
                                  rows[rb], gsem[rb]).wait()

        idx_start(0, 0)
        idx_start(1, 1)
        idx_wait(0)
        gather_start(0, 0)

        def body(k6, carry):
            for bb in range(6):
                kk = k6 * 6 + bb
                rb = bb % 2

                @pl.when(kk + 2 < nch)
                def _():
                    idx_start(kk + 2, (bb + 2) % 3)

                @pl.when(kk + 1 < nch)
                def _():
                    idx_wait((bb + 1) % 3)
                    gather_start((bb + 1) % 3, 1 - rb)

                gather_wait(bb % 3, rb)
                pltpu.sync_copy(rows[rb], acc_sh.at[dsts[bb % 3]], add=True)
            return carry

        lax.fori_loop(0, nch // 6, body, 0, unroll=False)
        plsc.subcore_barrier()

        rb = s * out_rows
        pltpu.sync_copy(acc_sh.at[pl.ds(rb, out_rows), pl.ds(0, d)],
                        agg_out.at[c, pl.ds(rb, out_rows)])
        pltpu.sync_copy(acc_sh.at[pl.ds(rb, out_rows), pl.ds(d, cw)],
                        cnt_out.at[c, pl.ds(rb, out_rows)])
        if tail:
            tb = NS * out_rows

            @pl.when(s == 0)
            def _():
                pltpu.sync_copy(acc_sh.at[pl.ds(tb, tail), pl.ds(0, d)],
                                agg_out.at[c, pl.ds(tb, tail)])
                pltpu.sync_copy(acc_sh.at[pl.ds(tb, tail), pl.ds(d, cw)],
                                cnt_out.at[c, pl.ds(tb, tail)])

    return k(hext, srcp, dstp, zrows)


def _gate_body(cnt_ref, oh_ref, oldz_ref, t1_ref, t2_ref, z_ref, gate_ref):
    counts = cnt_ref[0] + cnt_ref[1]
    oh = oh_ref[...]
    deg = jnp.sum(counts, axis=1, keepdims=True)
    match = jnp.sum(counts * oh, axis=1, keepdims=True)
    f1 = match / deg

    gc = jnp.sum(counts, axis=0, keepdims=True)
    present = gc > 0.0
    cnts_p = jnp.clip(counts / deg, 1e-5, None)
    ent = cnts_p * jnp.log(cnts_p)
    f2 = -jnp.sum(jnp.where(present, ent, 0.0), axis=1, keepdims=True)

    def _ln(x):
        m = jnp.mean(x)
        v = jnp.mean((x - m) ** 2)
        return (x - m) / jnp.sqrt(v + 1e-5)

    nf1 = _ln(f1)
    nf2 = _ln(f2)
    t1 = t1_ref[0, 0]
    t2 = t2_ref[0, 0]
    z = jax.nn.sigmoid(-(nf1 - t1)) * jax.nn.sigmoid(-(nf2 - t2))
    z_ref[...] = z
    gate_ref[...] = jnp.minimum(oldz_ref[...], z)


def _gates(cnt_part, oh, old_z, tau_1, tau_2, interpret=False):
    n = oh.shape[0]
    return pl.pallas_call(
        _gate_body,
        out_shape=(
            jax.ShapeDtypeStruct((n, 1), jnp.float32),
            jax.ShapeDtypeStruct((n, 1), jnp.float32),
        ),
        interpret=interpret,
    )(cnt_part, oh, old_z.reshape(n, 1), tau_1.reshape(1, 1),
      tau_2.reshape(1, 1))


def _newh_body(h_ref, agg_ref, gate_ref, norm_ref, out_ref):
    agg = agg_ref[0] + agg_ref[1]
    normagg = jax.nn.relu(agg * norm_ref[...])
    out_ref[...] = h_ref[...] + gate_ref[...] * normagg


def _new_h(h, agg_part, gate, norm, interpret=False):
    n, d = h.shape
    br = 1000 if n % 1000 == 0 else n
    grid = (n // br,)
    return pl.pallas_call(
        _newh_body,
        grid=grid,
        in_specs=[
            pl.BlockSpec((br, d), lambda i: (i, 0)),
            pl.BlockSpec((NC, br, d), lambda i: (0, i, 0)),
            pl.BlockSpec((br, 1), lambda i: (i, 0)),
            pl.BlockSpec((br, 1), lambda i: (i, 0)),
        ],
        out_specs=pl.BlockSpec((br, d), lambda i: (i, 0)),
        out_shape=jax.ShapeDtypeStruct((n, d), jnp.float32),
        interpret=interpret,
    )(h, agg_part, gate, norm.reshape(n, 1))


@jax.jit
def kernel(h, logits, old_z, norm, tau_1, tau_2, edge_index):
    n, d = h.shape
    c = logits.shape[1]
    cw = ((c + 15) // 16) * 16
    e = edge_index.shape[1]

    nt = NC * NS
    nch = -(-e // (nt * CHB))
    nch = -(-nch // 6) * 6
    ep = nt * CHB * nch
    pad = ep - e

    src = edge_index[0]
    dst = edge_index[1]
    if pad:
        pad_dst = n + (jnp.arange(pad, dtype=jnp.int32) % 8)
        src = jnp.concatenate([src, jnp.zeros((pad,), jnp.int32)])
        dst = jnp.concatenate([dst, pad_dst])

    rt = -(-(n + 8) // (NS * 8)) * NS * 8
    zrows = jnp.zeros((rt // NS, d + cw), jnp.float32)

    hext = _build_hext(h, logits, cw)
    agg_part, cnt_part = _edge_sc(n, d, cw, nch, rt, hext, src, dst, zrows)

    oh = lax.slice(hext, (0, d), (n, d + cw))
    z, gate = _gates(cnt_part, oh, old_z, tau_1, tau_2)
    new_h = _new_h(h, agg_part, gate, norm)
    return new_h, z.reshape(n)


# speedup vs baseline: 2.7527x; 1.3425x over previous
import functools

import jax
import jax.numpy as jnp
from jax import lax
from jax.experimental import pallas as pl
from jax.experimental.pallas import tpu as pltpu
from jax.experimental.pallas import tpu_sc as plsc

NC = 2
NS = 16
CHB = 48


def _hext_body(c, cw, logits_ref, h_ref, out_ref):
    lg = logits_ref[...]
    h = h_ref[...]
    m = jnp.max(lg, axis=1, keepdims=True)
    iota_c = lax.broadcasted_iota(jnp.int32, lg.shape, 1)
    cls = jnp.min(jnp.where(lg == m, iota_c, c), axis=1, keepdims=True)
    iota_w = lax.broadcasted_iota(jnp.int32, (lg.shape[0], cw), 1)
    oh = (iota_w == cls).astype(jnp.float32)
    out_ref[...] = jnp.concatenate([h, oh], axis=1)


def _build_hext(h, logits, cw, interpret=False):
    n, d = h.shape
    c = logits.shape[1]
    br = 1000 if n % 1000 == 0 else n
    grid = (n // br,)
    return pl.pallas_call(
        functools.partial(_hext_body, c, cw),
        grid=grid,
        in_specs=[
            pl.BlockSpec((br, c), lambda i: (i, 0)),
            pl.BlockSpec((br, d), lambda i: (i, 0)),
        ],
        out_specs=pl.BlockSpec((br, d + cw), lambda i: (i, 0)),
        out_shape=jax.ShapeDtypeStruct((n, d + cw), jnp.float32),
        interpret=interpret,
    )(logits, h)


def _edge_sc(n, d, cw, nch, rt, hext, srcp, dstp, zrows):
    w = d + cw
    rows_per_tile = rt // NS
    out_rows = (n // (NS * 8)) * 8
    tail = n - NS * out_rows

    mesh = plsc.VectorSubcoreMesh(core_axis_name="c", subcore_axis_name="s")

    @functools.partial(
        pl.kernel,
        out_type=(
            jax.ShapeDtypeStruct((NC, n, d), jnp.float32),
            jax.ShapeDtypeStruct((NC, n, cw), jnp.float32),
        ),
        mesh=mesh,
        scratch_types=[
            pltpu.VMEM_SHARED((rt, w), jnp.float32),
            [pltpu.VMEM((CHB, w), jnp.float32) for _ in range(2)],
            [pltpu.VMEM((CHB,), jnp.int32) for _ in range(3)],
            [pltpu.VMEM((CHB,), jnp.int32) for _ in range(3)],
            [pltpu.SemaphoreType.DMA for _ in range(3)],
            [pltpu.SemaphoreType.DMA for _ in range(2)],
        ],
        compiler_params=pltpu.CompilerParams(use_tc_tiling_on_sc=False),
    )
    def k(hext_hbm, src_hbm, dst_hbm, zrows_hbm,
          agg_out, cnt_out, acc_sh, rows, srcs, dsts, isem, gsem):
        c = lax.axis_index("c")
        s = lax.axis_index("s")
        wid = c * NS + s

        pltpu.sync_copy(zrows_hbm, acc_sh.at[pl.ds(s * rows_per_tile,
                                                   rows_per_tile)])
        plsc.subcore_barrier()

        ebase = wid * (nch * CHB)

        def idx_start(kk, r):
            off = pl.multiple_of(ebase + kk * CHB, CHB)
            pltpu.async_copy(src_hbm.at[pl.ds(off, CHB)], srcs[r], isem[r])
            pltpu.async_copy(dst_hbm.at[pl.ds(off, CHB)], dsts[r], isem[r])

        def idx_wait(r):
            pltpu.make_async_copy(src_hbm.at[pl.ds(0, CHB)],
                                  srcs[r], isem[r]).wait()
            pltpu.make_async_copy(dst_hbm.at[pl.ds(0, CHB)],
                                  dsts[r], isem[r]).wait()

        def gather_start(r, rb):
            pltpu.async_copy(hext_hbm.at[srcs[r]], rows[rb], gsem[rb])

        def gather_wait(r, rb):
            pltpu.make_async_copy(hext_hbm.at[srcs[r]],
                                  rows[rb], gsem[rb]).wait()

        idx_start(0, 0)
        idx_start(1, 1)
        idx_wait(0)
        gather_start(0, 0)

        def body(k6, carry):
            for bb in range(6):
                kk = k6 * 6 + bb
                rb = bb % 2

                @pl.when(kk + 2 < nch)
                def _():
                    idx_start(kk + 2, (bb + 2) % 3)

                @pl.when(kk + 1 < nch)
                def _():
                    idx_wait((bb + 1) % 3)
                    gather_start((bb + 1) % 3, 1 - rb)

                gather_wait(bb % 3, rb)
                pltpu.sync_copy(rows[rb], acc_sh.at[dsts[bb % 3]], add=True)
            return carry

        lax.fori_loop(0, nch // 6, body, 0, unroll=False)
        plsc.subcore_barrier()

        rb = s * out_rows
        pltpu.sync_copy(acc_sh.at[pl.ds(rb, out_rows), pl.ds(0, d)],
                        agg_out.at[c, pl.ds(rb, out_rows)])
        pltpu.sync_copy(acc_sh.at[pl.ds(rb, out_rows), pl.ds(d, cw)],
                        cnt_out.at[c, pl.ds(rb, out_rows)])
        if tail:
            tb = NS * out_rows

            @pl.when(s == 0)
            def _():
                pltpu.sync_copy(acc_sh.at[pl.ds(tb, tail), pl.ds(0, d)],
                                agg_out.at[c, pl.ds(tb, tail)])
                pltpu.sync_copy(acc_sh.at[pl.ds(tb, tail), pl.ds(d, cw)],
                                cnt_out.at[c, pl.ds(tb, tail)])

    return k(hext, srcp, dstp, zrows)


def _gate_body(cnt_ref, oh_ref, oldz_ref, t1_ref, t2_ref, z_ref, gate_ref):
    counts = cnt_ref[0] + cnt_ref[1]
    oh = oh_ref[...]
    deg = jnp.sum(counts, axis=1, keepdims=True)
    match = jnp.sum(counts * oh, axis=1, keepdims=True)
    f1 = match / deg

    gc = jnp.sum(counts, axis=0, keepdims=True)
    present = gc > 0.0
    cnts_p = jnp.clip(counts / deg, 1e-5, None)
    ent = cnts_p * jnp.log(cnts_p)
    f2 = -jnp.sum(jnp.where(present, ent, 0.0), axis=1, keepdims=True)

    def _ln(x):
        m = jnp.mean(x)
        v = jnp.mean((x - m) ** 2)
        return (x - m) / jnp.sqrt(v + 1e-5)

    nf1 = _ln(f1)
    nf2 = _ln(f2)
    t1 = t1_ref[0, 0]
    t2 = t2_ref[0, 0]
    z = jax.nn.sigmoid(-(nf1 - t1)) * jax.nn.sigmoid(-(nf2 - t2))
    z_ref[...] = z
    gate_ref[...] = jnp.minimum(oldz_ref[...], z)


def _gates(cnt_part, oh, old_z, tau_1, tau_2, interpret=False):
    n = oh.shape[0]
    return pl.pallas_call(
        _gate_body,
        out_shape=(
            jax.ShapeDtypeStruct((n, 1), jnp.float32),
            jax.ShapeDtypeStruct((n, 1), jnp.float32),
        ),
        interpret=interpret,
    )(cnt_part, oh, old_z.reshape(n, 1), tau_1.reshape(1, 1),
      tau_2.reshape(1, 1))


def _newh_body(h_ref, agg_ref, gate_ref, norm_ref, out_ref):
    agg = agg_ref[0] + agg_ref[1]
    normagg = jax.nn.relu(agg * norm_ref[...])
    out_ref[...] = h_ref[...] + gate_ref[...] * normagg


def _new_h(h, agg_part, gate, norm, interpret=False):
    n, d = h.shape
    br = 1000 if n % 1000 == 0 else n
    grid = (n // br,)
    return pl.pallas_call(
        _newh_body,
        grid=grid,
        in_specs=[
            pl.BlockSpec((br, d), lambda i: (i, 0)),
            pl.BlockSpec((NC, br, d), lambda i: (0, i, 0)),
            pl.BlockSpec((br, 1), lambda i: (i, 0)),
            pl.BlockSpec((br, 1), lambda i: (i, 0)),
        ],
        out_specs=pl.BlockSpec((br, d), lambda i: (i, 0)),
        out_shape=jax.ShapeDtypeStruct((n, d), jnp.float32),
        interpret=interpret,
    )(h, agg_part, gate, norm.reshape(n, 1))


@jax.jit
def kernel(h, logits, old_z, norm, tau_1, tau_2, edge_index):
    n, d = h.shape
    c = logits.shape[1]
    cw = ((c + 15) // 16) * 16
    e = edge_index.shape[1]

    nt = NC * NS
    nch = -(-e // (nt * CHB))
    nch = -(-nch // 6) * 6
    ep = nt * CHB * nch
    pad = ep - e

    src = edge_index[0]
    dst = edge_index[1]
    if pad:
        e2 = -(-e // nt) * nt
        if e2 != e:
            p0 = e2 - e
            r0 = jnp.arange(p0, dtype=jnp.int32)
            src = jnp.concatenate([src, (r0 * 37) % n])
            dst = jnp.concatenate([dst, n + r0 % 8])
        per = e2 // nt
        kpt = nch * CHB - per
        src = src.reshape(nt, per)
        dst = dst.reshape(nt, per)
        if kpt:
            rk = jnp.arange(kpt, dtype=jnp.int32)[None, :]
            rw = jnp.arange(nt, dtype=jnp.int32)[:, None]
            ps = ((rk + 37 * rw) * 131) % n
            pd = n + (rk + rw) % 8
            src = jnp.concatenate([src, jnp.broadcast_to(ps, (nt, kpt))],
                                  axis=1)
            dst = jnp.concatenate([dst, jnp.broadcast_to(pd, (nt, kpt))],
                                  axis=1)
        src = src.reshape(-1)
        dst = dst.reshape(-1)

    rt = -(-(n + 8) // (NS * 8)) * NS * 8
    zrows = jnp.zeros((rt // NS, d + cw), jnp.float32)

    hext = _build_hext(h, logits, cw)
    agg_part, cnt_part = _edge_sc(n, d, cw, nch, rt, hext, src, dst, zrows)

    oh = lax.slice(hext, (0, d), (n, d + cw))
    z, gate = _gates(cnt_part, oh, old_z, tau_1, tau_2)
    new_h = _new_h(h, agg_part, gate, norm)
    return new_h, z.reshape(n)
